# Initial kernel scaffold; baseline (speedup 1.0000x reference)
import jax
import jax.numpy as jnp
from jax.experimental import pallas as pl


def kernel(x, edge_index, W1, b1, W2, b2):
    raise NotImplementedError("write your pallas kernel here")


# speedup vs baseline: 11.4792x; 11.4792x over previous
import functools

import jax
import jax.numpy as jnp
from jax import lax
from jax.experimental import pallas as pl
from jax.experimental.pallas import tpu as pltpu
from jax.experimental.pallas import tpu_sc as plsc

_N = 10000
_D = 128
_OD = 2
_NP = 10240
_NC, _NS = 2, 16
_NW = _NC * _NS
_CH = 128
_E = 320000
_K = 80
_EP = _NW * _K * _CH
_OW = 16
_RB = 512
_RPS = _NP // _NS


def _mesh():
    return plsc.VectorSubcoreMesh(core_axis_name="c", subcore_axis_name="s",
                                  num_cores=_NC, num_subcores=_NS)


def _deg_sc(dst3, z1, one):
    @functools.partial(
        pl.kernel,
        out_type=jax.ShapeDtypeStruct((_NC, _NP), jnp.float32),
        mesh=_mesh(),
        scratch_types=[
            pltpu.VMEM((_K, _CH), jnp.int32),
            pltpu.VMEM((_CH,), jnp.float32),
            pltpu.VMEM_SHARED((_NP,), jnp.float32),
        ],
    )
    def k(dst_hbm, z_hbm, one_hbm, out_hbm, idx_v, ones_v, deg_sh):
        c = lax.axis_index("c")
        s = lax.axis_index("s")
        w = c * _NS + s
        pltpu.sync_copy(dst_hbm.at[w], idx_v)
        pltpu.sync_copy(one_hbm, ones_v)
        sl = pl.ds(s * _RPS, _RPS)
        pltpu.sync_copy(z_hbm.at[sl], deg_sh.at[sl])
        plsc.subcore_barrier()

        def step(j, carry):
            pltpu.sync_copy(ones_v, deg_sh.at[idx_v.at[j]], add=True)
            return carry

        lax.fori_loop(0, _K, step, 0)
        plsc.subcore_barrier()
        pltpu.sync_copy(deg_sh.at[sl], out_hbm.at[c, sl])

    return k(dst3, z1, one)


def _scatter_sc(src3, dst3, feat, zeros, width):
    @functools.partial(
        pl.kernel,
        out_type=jax.ShapeDtypeStruct((_NC, _NP, width), jnp.float32),
        mesh=_mesh(),
        compiler_params=pltpu.CompilerParams(
            use_tc_tiling_on_sc=(width == _D)),
        scratch_types=[
            pltpu.VMEM((_K, _CH), jnp.int32),
            pltpu.VMEM((_K, _CH), jnp.int32),
            pltpu.VMEM((_CH, width), jnp.float32),
            pltpu.SemaphoreType.DMA,
            pltpu.VMEM_SHARED((_NP, width), jnp.float32),
        ],
    )
    def k(src_hbm, dst_hbm, feat_hbm, z_hbm, out_hbm,
          isrc_v, idst_v, rows_v, sem, acc_sh):
        c = lax.axis_index("c")
        s = lax.axis_index("s")
        w = c * _NS + s
        pltpu.sync_copy(src_hbm.at[w], isrc_v)
        pltpu.sync_copy(dst_hbm.at[w], idst_v)
        sl = pl.ds(s * _RPS, _RPS)
        pltpu.sync_copy(z_hbm.at[sl], acc_sh.at[sl])
        plsc.subcore_barrier()

        def step(j, carry):
            pltpu.async_copy(feat_hbm.at[isrc_v.at[j]], rows_v, sem).wait()
            pltpu.sync_copy(rows_v, acc_sh.at[idst_v.at[j]], add=True)
            return carry

        lax.fori_loop(0, _K, step, 0)
        plsc.subcore_barrier()
        pltpu.sync_copy(acc_sh.at[sl], out_hbm.at[c, sl])

    return k(src3, dst3, feat, zeros)


def _mm_scale_tc(xp, W1, degT):
    def body(x_ref, w_ref, deg_ref, hs_ref):
        h = jnp.dot(x_ref[...], w_ref[...], preferred_element_type=jnp.float32)
        dsum = deg_ref[:, 0:1] + deg_ref[:, 1:2]
        dinv = lax.rsqrt(dsum + 1.0)
        hs_ref[...] = h * dinv

    return pl.pallas_call(
        body,
        grid=(_NP // _RB,),
        in_specs=[
            pl.BlockSpec((_RB, _D), lambda i: (i, 0)),
            pl.BlockSpec((_D, _D), lambda i: (0, 0)),
            pl.BlockSpec((_RB, _NC), lambda i: (i, 0)),
        ],
        out_specs=pl.BlockSpec((_RB, _D), lambda i: (i, 0)),
        out_shape=jax.ShapeDtypeStruct((_NP, _D), jnp.float32),
    )(xp, W1, degT)


def _layer2_tc(p0, p1, hs, degT, W2p, b1r):
    def body(p0_ref, p1_ref, hs_ref, deg_ref, w2_ref, b1_ref, qs_ref):
        dsum = deg_ref[:, 0:1] + deg_ref[:, 1:2]
        dinv = lax.rsqrt(dsum + 1.0)
        pre = (p0_ref[...] + p1_ref[...] + hs_ref[...]) * dinv + b1_ref[...]
        xb1 = jnp.maximum(pre, 0.0)
        q = jnp.dot(xb1, w2_ref[...], preferred_element_type=jnp.float32)
        qs_ref[...] = q * dinv

    return pl.pallas_call(
        body,
        grid=(_NP // _RB,),
        in_specs=[
            pl.BlockSpec((_RB, _D), lambda i: (i, 0)),
            pl.BlockSpec((_RB, _D), lambda i: (i, 0)),
            pl.BlockSpec((_RB, _D), lambda i: (i, 0)),
            pl.BlockSpec((_RB, _NC), lambda i: (i, 0)),
            pl.BlockSpec((_D, _OW), lambda i: (0, 0)),
            pl.BlockSpec((1, _D), lambda i: (0, 0)),
        ],
        out_specs=pl.BlockSpec((_RB, _OW), lambda i: (i, 0)),
        out_shape=jax.ShapeDtypeStruct((_NP, _OW), jnp.float32),
    )(p0, p1, hs, degT, W2p, b1r)


def _final_tc(q0, q1, qs, degT, b2r):
    def body(q0_ref, q1_ref, qs_ref, deg_ref, b2_ref, out_ref):
        dsum = deg_ref[:, 0:1] + deg_ref[:, 1:2]
        dinv = lax.rsqrt(dsum + 1.0)
        out_ref[...] = (q0_ref[...] + q1_ref[...] + qs_ref[...]) * dinv + b2_ref[...]

    return pl.pallas_call(
        body,
        grid=(_NP // _RB,),
        in_specs=[
            pl.BlockSpec((_RB, _OW), lambda i: (i, 0)),
            pl.BlockSpec((_RB, _OW), lambda i: (i, 0)),
            pl.BlockSpec((_RB, _OW), lambda i: (i, 0)),
            pl.BlockSpec((_RB, _NC), lambda i: (i, 0)),
            pl.BlockSpec((1, _OW), lambda i: (0, 0)),
        ],
        out_specs=pl.BlockSpec((_RB, _OW), lambda i: (i, 0)),
        out_shape=jax.ShapeDtypeStruct((_NP, _OW), jnp.float32),
    )(q0, q1, qs, degT, b2r)


def kernel(x, edge_index, W1, b1, W2, b2):
    ei = edge_index.astype(jnp.int32)
    pad = jnp.full((_EP - _E,), _NP - 1, jnp.int32)
    src3 = jnp.concatenate([ei[0], pad]).reshape(_NW, _K, _CH)
    dst3 = jnp.concatenate([ei[1], pad]).reshape(_NW, _K, _CH)
    xp = jnp.zeros((_NP, _D), jnp.float32).at[:_N].set(x)
    z1 = jnp.zeros((_NP,), jnp.float32)
    zD = jnp.zeros((_NP, _D), jnp.float32)
    zO = jnp.zeros((_NP, _OW), jnp.float32)
    one = jnp.ones((_CH,), jnp.float32)
    W2p = jnp.zeros((_D, _OW), jnp.float32).at[:, :_OD].set(W2)
    b1r = b1.reshape(1, _D)
    b2r = jnp.zeros((1, _OW), jnp.float32).at[0, :_OD].set(b2)

    degs = _deg_sc(dst3, z1, one)
    degT = degs.T
    hs = _mm_scale_tc(xp, W1, degT)
    parts = _scatter_sc(src3, dst3, hs, zD, _D)
    qs = _layer2_tc(parts[0], parts[1], hs, degT, W2p, b1r)
    parts2 = _scatter_sc(src3, dst3, qs, zO, _OW)
    outf = _final_tc(parts2[0], parts2[1], qs, degT, b2r)
    return outf[:_N, :_OD]


# speedup vs baseline: 24.9081x; 2.1698x over previous
import functools

import jax
import jax.numpy as jnp
from jax import lax
from jax.experimental import pallas as pl
from jax.experimental.pallas import tpu as pltpu
from jax.experimental.pallas import tpu_sc as plsc

_N = 10000
_D = 128
_OD = 2
_NP = 10240
_NC, _NS = 2, 16
_NW = _NC * _NS
_CH = 128
_E = 320000
_K = 80
_EP = _NW * _K * _CH
_OW = 16
_RB = 512
_RPS = _NP // _NS


def _mesh():
    return plsc.VectorSubcoreMesh(core_axis_name="c", subcore_axis_name="s",
                                  num_cores=_NC, num_subcores=_NS)


def _deg_sc(dst3, z1, one):
    @functools.partial(
        pl.kernel,
        out_type=jax.ShapeDtypeStruct((_NC, _NP), jnp.float32),
        mesh=_mesh(),
        scratch_types=[
            pltpu.VMEM((_K, _CH), jnp.int32),
            pltpu.VMEM((_CH,), jnp.float32),
            pltpu.VMEM_SHARED((_NP,), jnp.float32),
        ],
    )
    def k(dst_hbm, z_hbm, one_hbm, out_hbm, idx_v, ones_v, deg_sh):
        c = lax.axis_index("c")
        s = lax.axis_index("s")
        w = c * _NS + s
        pltpu.sync_copy(dst_hbm.at[w], idx_v)
        pltpu.sync_copy(one_hbm, ones_v)
        sl = pl.ds(s * _RPS, _RPS)
        pltpu.sync_copy(z_hbm.at[sl], deg_sh.at[sl])
        plsc.subcore_barrier()

        def step(j, carry):
            pltpu.sync_copy(ones_v, deg_sh.at[idx_v.at[j]], add=True)
            return carry

        lax.fori_loop(0, _K, step, 0)
        plsc.subcore_barrier()
        pltpu.sync_copy(deg_sh.at[sl], out_hbm.at[c, sl])

    return k(dst3, z1, one)


def _scatter_sc(src3, dst3, feat, zeros, width):
    @functools.partial(
        pl.kernel,
        out_type=jax.ShapeDtypeStruct((_NC, _NP, width), jnp.float32),
        mesh=_mesh(),
        compiler_params=pltpu.CompilerParams(
            use_tc_tiling_on_sc=(width == _D)),
        scratch_types=[
            pltpu.VMEM((_K, _CH), jnp.int32),
            pltpu.VMEM((_K, _CH), jnp.int32),
            pltpu.VMEM((_CH, width), jnp.float32),
            pltpu.SemaphoreType.DMA,
            pltpu.VMEM_SHARED((_NP, width), jnp.float32),
        ],
    )
    def k(src_hbm, dst_hbm, feat_hbm, z_hbm, out_hbm,
          isrc_v, idst_v, rows_v, sem, acc_sh):
        c = lax.axis_index("c")
        s = lax.axis_index("s")
        w = c * _NS + s
        pltpu.sync_copy(src_hbm.at[w], isrc_v)
        pltpu.sync_copy(dst_hbm.at[w], idst_v)
        sl = pl.ds(s * _RPS, _RPS)
        pltpu.sync_copy(z_hbm.at[sl], acc_sh.at[sl])
        plsc.subcore_barrier()

        def step(j, carry):
            pltpu.async_copy(feat_hbm.at[isrc_v.at[j]], rows_v, sem).wait()
            pltpu.sync_copy(rows_v, acc_sh.at[idst_v.at[j]], add=True)
            return carry

        lax.fori_loop(0, _K, step, 0)
        plsc.subcore_barrier()
        pltpu.sync_copy(acc_sh.at[sl], out_hbm.at[c, sl])

    return k(src3, dst3, feat, zeros)


def _mm_scale_tc(xp, W1, degT):
    def body(x_ref, w_ref, deg_ref, hs_ref):
        h = jnp.dot(x_ref[...], w_ref[...], preferred_element_type=jnp.float32)
        dsum = deg_ref[:, 0:1] + deg_ref[:, 1:2]
        dinv = lax.rsqrt(dsum + 1.0)
        hs_ref[...] = h * dinv

    return pl.pallas_call(
        body,
        grid=(_NP // _RB,),
        in_specs=[
            pl.BlockSpec((_RB, _D), lambda i: (i, 0)),
            pl.BlockSpec((_D, _D), lambda i: (0, 0)),
            pl.BlockSpec((_RB, _NC), lambda i: (i, 0)),
        ],
        out_specs=pl.BlockSpec((_RB, _D), lambda i: (i, 0)),
        out_shape=jax.ShapeDtypeStruct((_NP, _D), jnp.float32),
    )(xp, W1, degT)


def _layer2_tc(p0, p1, hs, degT, W2p, b1r):
    def body(p0_ref, p1_ref, hs_ref, deg_ref, w2_ref, b1_ref, qs_ref):
        dsum = deg_ref[:, 0:1] + deg_ref[:, 1:2]
        dinv = lax.rsqrt(dsum + 1.0)
        pre = (p0_ref[...] + p1_ref[...] + hs_ref[...]) * dinv + b1_ref[...]
        xb1 = jnp.maximum(pre, 0.0)
        q = jnp.dot(xb1, w2_ref[...], preferred_element_type=jnp.float32)
        qs_ref[...] = q * dinv

    return pl.pallas_call(
        body,
        grid=(_NP // _RB,),
        in_specs=[
            pl.BlockSpec((_RB, _D), lambda i: (i, 0)),
            pl.BlockSpec((_RB, _D), lambda i: (i, 0)),
            pl.BlockSpec((_RB, _D), lambda i: (i, 0)),
            pl.BlockSpec((_RB, _NC), lambda i: (i, 0)),
            pl.BlockSpec((_D, _OW), lambda i: (0, 0)),
            pl.BlockSpec((1, _D), lambda i: (0, 0)),
        ],
        out_specs=pl.BlockSpec((_RB, _OW), lambda i: (i, 0)),
        out_shape=jax.ShapeDtypeStruct((_NP, _OW), jnp.float32),
    )(p0, p1, hs, degT, W2p, b1r)


def _final_tc(q0, q1, qs, degT, b2r):
    def body(q0_ref, q1_ref, qs_ref, deg_ref, b2_ref, out_ref):
        dsum = deg_ref[:, 0:1] + deg_ref[:, 1:2]
        dinv = lax.rsqrt(dsum + 1.0)
        out_ref[...] = (q0_ref[...] + q1_ref[...] + qs_ref[...]) * dinv + b2_ref[...]

    return pl.pallas_call(
        body,
        grid=(_NP // _RB,),
        in_specs=[
            pl.BlockSpec((_RB, _OW), lambda i: (i, 0)),
            pl.BlockSpec((_RB, _OW), lambda i: (i, 0)),
            pl.BlockSpec((_RB, _OW), lambda i: (i, 0)),
            pl.BlockSpec((_RB, _NC), lambda i: (i, 0)),
            pl.BlockSpec((1, _OW), lambda i: (0, 0)),
        ],
        out_specs=pl.BlockSpec((_RB, _OW), lambda i: (i, 0)),
        out_shape=jax.ShapeDtypeStruct((_NP, _OW), jnp.float32),
    )(q0, q1, qs, degT, b2r)


def kernel(x, edge_index, W1, b1, W2, b2):
    ei = edge_index.astype(jnp.int32)
    pad = _N + (jnp.arange(_EP - _E, dtype=jnp.int32) % (_NP - _N))
    src3 = jnp.concatenate([ei[0], pad]).reshape(_NW, _K, _CH)
    dst3 = jnp.concatenate([ei[1], pad]).reshape(_NW, _K, _CH)
    xp = jnp.zeros((_NP, _D), jnp.float32).at[:_N].set(x)
    z1 = jnp.zeros((_NP,), jnp.float32)
    zD = jnp.zeros((_NP, _D), jnp.float32)
    zO = jnp.zeros((_NP, _OW), jnp.float32)
    one = jnp.ones((_CH,), jnp.float32)
    W2p = jnp.zeros((_D, _OW), jnp.float32).at[:, :_OD].set(W2)
    b1r = b1.reshape(1, _D)
    b2r = jnp.zeros((1, _OW), jnp.float32).at[0, :_OD].set(b2)

    degs = _deg_sc(dst3, z1, one)
    degT = degs.T
    hs = _mm_scale_tc(xp, W1, degT)
    parts = _scatter_sc(src3, dst3, hs, zD, _D)
    qs = _layer2_tc(parts[0], parts[1], hs, degT, W2p, b1r)
    parts2 = _scatter_sc(src3, dst3, qs, zO, _OW)
    outf = _final_tc(parts2[0], parts2[1], qs, degT, b2r)
    return outf[:_N, :_OD]


# speedup vs baseline: 33.9990x; 1.3650x over previous
import functools

import jax
import jax.numpy as jnp
from jax import lax
from jax.experimental import pallas as pl
from jax.experimental.pallas import tpu as pltpu
from jax.experimental.pallas import tpu_sc as plsc

_N = 10000
_D = 128
_OD = 2
_NP = 10240
_NC, _NS = 2, 16
_NW = _NC * _NS
_CH = 128
_E = 320000
_K = 80
_EP = _NW * _K * _CH
_OW = 16
_RB = 512
_RPS = _NP // _NS


def _mesh():
    return plsc.VectorSubcoreMesh(core_axis_name="c", subcore_axis_name="s",
                                  num_cores=_NC, num_subcores=_NS)


def _deg_sc(dst3, z1, one):
    @functools.partial(
        pl.kernel,
        out_type=jax.ShapeDtypeStruct((_NC, _NP), jnp.float32),
        mesh=_mesh(),
        scratch_types=[
            pltpu.VMEM((_K, _CH), jnp.int32),
            pltpu.VMEM((_CH,), jnp.float32),
            pltpu.VMEM_SHARED((_NP,), jnp.float32),
        ],
    )
    def k(dst_hbm, z_hbm, one_hbm, out_hbm, idx_v, ones_v, deg_sh):
        c = lax.axis_index("c")
        s = lax.axis_index("s")
        w = c * _NS + s
        pltpu.sync_copy(dst_hbm.at[w], idx_v)
        pltpu.sync_copy(one_hbm, ones_v)
        sl = pl.ds(s * _RPS, _RPS)
        pltpu.sync_copy(z_hbm.at[sl], deg_sh.at[sl])
        plsc.subcore_barrier()

        def step(j, carry):
            pltpu.sync_copy(ones_v, deg_sh.at[idx_v.at[j]], add=True)
            return carry

        lax.fori_loop(0, _K, step, 0)
        plsc.subcore_barrier()
        pltpu.sync_copy(deg_sh.at[sl], out_hbm.at[c, sl])

    return k(dst3, z1, one)


def _scatter_sc(src3, dst3, feat, zeros, width):
    @functools.partial(
        pl.kernel,
        out_type=jax.ShapeDtypeStruct((_NC, _NP, width), jnp.float32),
        mesh=_mesh(),
        compiler_params=pltpu.CompilerParams(
            use_tc_tiling_on_sc=(width == _D)),
        scratch_types=[
            pltpu.VMEM((_K, _CH), jnp.int32),
            pltpu.VMEM((_CH,), jnp.int32),
            pltpu.VMEM((_CH,), jnp.int32),
            pltpu.VMEM((_CH, width), jnp.float32),
            pltpu.VMEM((_CH, width), jnp.float32),
            pltpu.SemaphoreType.DMA,
            pltpu.SemaphoreType.DMA,
            pltpu.SemaphoreType.DMA,
            pltpu.SemaphoreType.DMA,
            pltpu.VMEM_SHARED((_NP, width), jnp.float32),
        ],
    )
    def k(src_hbm, dst_hbm, feat_hbm, z_hbm, out_hbm,
          isrc_v, id0_v, id1_v, rows0_v, rows1_v,
          gsem0, gsem1, dsem0, dsem1, acc_sh):
        c = lax.axis_index("c")
        s = lax.axis_index("s")
        w = c * _NS + s
        pltpu.sync_copy(src_hbm.at[w], isrc_v)
        sl = pl.ds(s * _RPS, _RPS)
        pltpu.sync_copy(z_hbm.at[sl], acc_sh.at[sl])
        plsc.subcore_barrier()

        pltpu.async_copy(dst_hbm.at[w, 0], id0_v, dsem0)
        pltpu.async_copy(dst_hbm.at[w, 1], id1_v, dsem1)
        pltpu.async_copy(feat_hbm.at[isrc_v.at[0]], rows0_v, gsem0)
        pltpu.async_copy(feat_hbm.at[isrc_v.at[1]], rows1_v, gsem1)

        def half(j, id_v, rows_v, gsem, dsem):
            pltpu.make_async_copy(dst_hbm.at[w, j], id_v, dsem).wait()
            pltpu.make_async_copy(feat_hbm.at[isrc_v.at[j]], rows_v, gsem).wait()
            pltpu.sync_copy(rows_v, acc_sh.at[id_v], add=True)

            @pl.when(j + 2 < _K)
            def _():
                pltpu.async_copy(dst_hbm.at[w, j + 2], id_v, dsem)
                pltpu.async_copy(feat_hbm.at[isrc_v.at[j + 2]], rows_v, gsem)

        def step(jj, carry):
            j0 = 2 * jj
            half(j0, id0_v, rows0_v, gsem0, dsem0)
            half(j0 + 1, id1_v, rows1_v, gsem1, dsem1)
            return carry

        lax.fori_loop(0, _K // 2, step, 0)
        plsc.subcore_barrier()
        pltpu.sync_copy(acc_sh.at[sl], out_hbm.at[c, sl])

    return k(src3, dst3, feat, zeros)


def _mm_scale_tc(xp, W1, degT):
    def body(x_ref, w_ref, deg_ref, hs_ref):
        h = jnp.dot(x_ref[...], w_ref[...], preferred_element_type=jnp.float32)
        dsum = deg_ref[:, 0:1] + deg_ref[:, 1:2]
        dinv = lax.rsqrt(dsum + 1.0)
        hs_ref[...] = h * dinv

    return pl.pallas_call(
        body,
        grid=(_NP // _RB,),
        in_specs=[
            pl.BlockSpec((_RB, _D), lambda i: (i, 0)),
            pl.BlockSpec((_D, _D), lambda i: (0, 0)),
            pl.BlockSpec((_RB, _NC), lambda i: (i, 0)),
        ],
        out_specs=pl.BlockSpec((_RB, _D), lambda i: (i, 0)),
        out_shape=jax.ShapeDtypeStruct((_NP, _D), jnp.float32),
    )(xp, W1, degT)


def _layer2_tc(p0, p1, hs, degT, W2p, b1r):
    def body(p0_ref, p1_ref, hs_ref, deg_ref, w2_ref, b1_ref, qs_ref):
        dsum = deg_ref[:, 0:1] + deg_ref[:, 1:2]
        dinv = lax.rsqrt(dsum + 1.0)
        pre = (p0_ref[...] + p1_ref[...] + hs_ref[...]) * dinv + b1_ref[...]
        xb1 = jnp.maximum(pre, 0.0)
        q = jnp.dot(xb1, w2_ref[...], preferred_element_type=jnp.float32)
        qs_ref[...] = q * dinv

    return pl.pallas_call(
        body,
        grid=(_NP // _RB,),
        in_specs=[
            pl.BlockSpec((_RB, _D), lambda i: (i, 0)),
            pl.BlockSpec((_RB, _D), lambda i: (i, 0)),
            pl.BlockSpec((_RB, _D), lambda i: (i, 0)),
            pl.BlockSpec((_RB, _NC), lambda i: (i, 0)),
            pl.BlockSpec((_D, _OW), lambda i: (0, 0)),
            pl.BlockSpec((1, _D), lambda i: (0, 0)),
        ],
        out_specs=pl.BlockSpec((_RB, _OW), lambda i: (i, 0)),
        out_shape=jax.ShapeDtypeStruct((_NP, _OW), jnp.float32),
    )(p0, p1, hs, degT, W2p, b1r)


def _final_tc(q0, q1, qs, degT, b2r):
    def body(q0_ref, q1_ref, qs_ref, deg_ref, b2_ref, out_ref):
        dsum = deg_ref[:, 0:1] + deg_ref[:, 1:2]
        dinv = lax.rsqrt(dsum + 1.0)
        out_ref[...] = (q0_ref[...] + q1_ref[...] + qs_ref[...]) * dinv + b2_ref[...]

    return pl.pallas_call(
        body,
        grid=(_NP // _RB,),
        in_specs=[
            pl.BlockSpec((_RB, _OW), lambda i: (i, 0)),
            pl.BlockSpec((_RB, _OW), lambda i: (i, 0)),
            pl.BlockSpec((_RB, _OW), lambda i: (i, 0)),
            pl.BlockSpec((_RB, _NC), lambda i: (i, 0)),
            pl.BlockSpec((1, _OW), lambda i: (0, 0)),
        ],
        out_specs=pl.BlockSpec((_RB, _OW), lambda i: (i, 0)),
        out_shape=jax.ShapeDtypeStruct((_NP, _OW), jnp.float32),
    )(q0, q1, qs, degT, b2r)


def kernel(x, edge_index, W1, b1, W2, b2):
    ei = edge_index.astype(jnp.int32)
    pad = _N + (jnp.arange(_EP - _E, dtype=jnp.int32) % (_NP - _N))
    src3 = jnp.concatenate([ei[0], pad]).reshape(_NW, _K, _CH)
    dst3 = jnp.concatenate([ei[1], pad]).reshape(_NW, _K, _CH)
    xp = jnp.zeros((_NP, _D), jnp.float32).at[:_N].set(x)
    z1 = jnp.zeros((_NP,), jnp.float32)
    zD = jnp.zeros((_NP, _D), jnp.float32)
    zO = jnp.zeros((_NP, _OW), jnp.float32)
    one = jnp.ones((_CH,), jnp.float32)
    W2p = jnp.zeros((_D, _OW), jnp.float32).at[:, :_OD].set(W2)
    b1r = b1.reshape(1, _D)
    b2r = jnp.zeros((1, _OW), jnp.float32).at[0, :_OD].set(b2)

    degs = _deg_sc(dst3, z1, one)
    degT = degs.T
    hs = _mm_scale_tc(xp, W1, degT)
    parts = _scatter_sc(src3, dst3, hs, zD, _D)
    qs = _layer2_tc(parts[0], parts[1], hs, degT, W2p, b1r)
    parts2 = _scatter_sc(src3, dst3, qs, zO, _OW)
    outf = _final_tc(parts2[0], parts2[1], qs, degT, b2r)
    return outf[:_N, :_OD]
